# baseline (speedup 1.0000x reference)
import functools

import jax
import jax.numpy as jnp
from jax import lax
from jax.experimental import pallas as pl
from jax.experimental.pallas import tpu as pltpu
from jax.experimental.pallas import tpu_sc as plsc

D_MODEL = 768
DPART = 256
S_TOTAL = 4096
S_BLK = 512
N_SBLK = S_TOTAL // S_BLK
NW = 32
S_PER_W = S_TOTAL // NW


def _sc_gather(ix, iy, iz, Wx, Wy, Wz):
    mesh = plsc.VectorSubcoreMesh(core_axis_name="c", subcore_axis_name="s")
    out_t = jax.ShapeDtypeStruct((S_TOTAL, DPART), jnp.float32)

    @functools.partial(
        pl.kernel,
        out_type=(out_t, out_t, out_t),
        mesh=mesh,
        scratch_types=[
            pltpu.VMEM((S_PER_W,), jnp.int32),
            pltpu.VMEM((S_PER_W, DPART), jnp.float32),
            pltpu.SemaphoreType.DMA,
        ],
    )
    def k(ix_hbm, iy_hbm, iz_hbm, wx_hbm, wy_hbm, wz_hbm,
          ox_hbm, oy_hbm, oz_hbm, idx_v, rows_v, sem):
        wid = lax.axis_index("s") * 2 + lax.axis_index("c")
        base = wid * S_PER_W
        for ih, wh, oh in ((ix_hbm, wx_hbm, ox_hbm),
                           (iy_hbm, wy_hbm, oy_hbm),
                           (iz_hbm, wz_hbm, oz_hbm)):
            pltpu.sync_copy(ih.at[pl.ds(base, S_PER_W)], idx_v)
            pltpu.async_copy(wh.at[idx_v], rows_v, sem).wait()
            pltpu.sync_copy(rows_v, oh.at[pl.ds(base, S_PER_W)])

    return k(ix, iy, iz, Wx, Wy, Wz)


def _add_body(x_ref, ex_ref, ey_ref, ez_ref, o_ref):
    xb = x_ref[0]
    o_ref[0, :, 0:DPART] = xb[:, 0:DPART] + ex_ref[...]
    o_ref[0, :, DPART:2 * DPART] = xb[:, DPART:2 * DPART] + ey_ref[...]
    o_ref[0, :, 2 * DPART:D_MODEL] = xb[:, 2 * DPART:D_MODEL] + ez_ref[...]


def kernel(x, src_tgt, src_pos_x, src_pos_y, src_pos_z, Wx, Wy, Wz):
    del src_tgt
    B = x.shape[0]
    ex, ey, ez = _sc_gather(src_pos_x, src_pos_y, src_pos_z, Wx, Wy, Wz)

    e_spec = pl.BlockSpec((S_BLK, DPART), lambda i, j: (i, 0))
    x_spec = pl.BlockSpec((1, S_BLK, D_MODEL), lambda i, j: (j, i, 0))

    return pl.pallas_call(
        _add_body,
        grid=(N_SBLK, B),
        in_specs=[x_spec, e_spec, e_spec, e_spec],
        out_specs=x_spec,
        out_shape=jax.ShapeDtypeStruct(x.shape, x.dtype),
    )(x, ex, ey, ez)


# speedup vs baseline: 3.1733x; 3.1733x over previous
import jax
import jax.numpy as jnp
from jax import lax
from jax.experimental import pallas as pl

D_MODEL = 768
DPART = 256
S_TOTAL = 4096
S_BLK = 1024
N_SBLK = S_TOTAL // S_BLK


def _body(ix_ref, iy_ref, iz_ref, x_ref, wx_ref, wy_ref, wz_ref, o_ref):
    iota = lax.broadcasted_iota(jnp.int32, (32, S_BLK), 0)

    def part(idx_ref, w_ref):
        oh = (idx_ref[0, 0, :][None, :] == iota).astype(jnp.float32)
        return lax.dot_general(
            oh, w_ref[...], (((0,), (0,)), ((), ())),
            preferred_element_type=jnp.float32,
        )

    ex = part(ix_ref, wx_ref)
    ey = part(iy_ref, wy_ref)
    ez = part(iz_ref, wz_ref)
    xb = x_ref[0]
    o_ref[0, :, 0:DPART] = xb[:, 0:DPART] + ex
    o_ref[0, :, DPART:2 * DPART] = xb[:, DPART:2 * DPART] + ey
    o_ref[0, :, 2 * DPART:D_MODEL] = xb[:, 2 * DPART:D_MODEL] + ez


def kernel(x, src_tgt, src_pos_x, src_pos_y, src_pos_z, Wx, Wy, Wz):
    del src_tgt
    B = x.shape[0]
    ix = src_pos_x.reshape(N_SBLK, 1, S_BLK)
    iy = src_pos_y.reshape(N_SBLK, 1, S_BLK)
    iz = src_pos_z.reshape(N_SBLK, 1, S_BLK)

    idx_spec = pl.BlockSpec((1, 1, S_BLK), lambda i, j: (i, 0, 0))
    tab_spec = pl.BlockSpec((32, DPART), lambda i, j: (0, 0))
    x_spec = pl.BlockSpec((1, S_BLK, D_MODEL), lambda i, j: (j, i, 0))

    return pl.pallas_call(
        _body,
        grid=(N_SBLK, B),
        in_specs=[idx_spec, idx_spec, idx_spec, x_spec, tab_spec, tab_spec,
                  tab_spec],
        out_specs=x_spec,
        out_shape=jax.ShapeDtypeStruct(x.shape, x.dtype),
    )(ix, iy, iz, x, Wx, Wy, Wz)


# speedup vs baseline: 3.3820x; 1.0658x over previous
import jax
import jax.numpy as jnp
from jax import lax
from jax.experimental import pallas as pl

D_MODEL = 768
DPART = 256
S_TOTAL = 4096
S_BLK = 2048
N_SBLK = S_TOTAL // S_BLK


def _body(ix_ref, iy_ref, iz_ref, x_ref, wx_ref, wy_ref, wz_ref, o_ref):
    iota = lax.broadcasted_iota(jnp.int32, (32, S_BLK), 0)

    def part(idx_ref, w_ref):
        oh = (idx_ref[0, 0, :][None, :] == iota).astype(jnp.float32)
        return lax.dot_general(
            oh, w_ref[...], (((0,), (0,)), ((), ())),
            preferred_element_type=jnp.float32,
        )

    ex = part(ix_ref, wx_ref)
    ey = part(iy_ref, wy_ref)
    ez = part(iz_ref, wz_ref)
    xb = x_ref[0]
    o_ref[0, :, 0:DPART] = xb[:, 0:DPART] + ex
    o_ref[0, :, DPART:2 * DPART] = xb[:, DPART:2 * DPART] + ey
    o_ref[0, :, 2 * DPART:D_MODEL] = xb[:, 2 * DPART:D_MODEL] + ez


def kernel(x, src_tgt, src_pos_x, src_pos_y, src_pos_z, Wx, Wy, Wz):
    del src_tgt
    B = x.shape[0]
    ix = src_pos_x.reshape(N_SBLK, 1, S_BLK)
    iy = src_pos_y.reshape(N_SBLK, 1, S_BLK)
    iz = src_pos_z.reshape(N_SBLK, 1, S_BLK)

    idx_spec = pl.BlockSpec((1, 1, S_BLK), lambda i, j: (i, 0, 0))
    tab_spec = pl.BlockSpec((32, DPART), lambda i, j: (0, 0))
    x_spec = pl.BlockSpec((1, S_BLK, D_MODEL), lambda i, j: (j, i, 0))

    return pl.pallas_call(
        _body,
        grid=(N_SBLK, B),
        in_specs=[idx_spec, idx_spec, idx_spec, x_spec, tab_spec, tab_spec,
                  tab_spec],
        out_specs=x_spec,
        out_shape=jax.ShapeDtypeStruct(x.shape, x.dtype),
    )(ix, iy, iz, x, Wx, Wy, Wz)


# speedup vs baseline: 3.6164x; 1.0693x over previous
import jax
import jax.numpy as jnp
from jax import lax
from jax.experimental import pallas as pl

D_MODEL = 768
DPART = 256
S_TOTAL = 4096
S_BLK = 4096
N_SBLK = S_TOTAL // S_BLK


def _body(ix_ref, iy_ref, iz_ref, x_ref, wx_ref, wy_ref, wz_ref, o_ref):
    iota = lax.broadcasted_iota(jnp.int32, (32, S_BLK), 0)

    def part(idx_ref, w_ref):
        oh = (idx_ref[0, 0, :][None, :] == iota).astype(jnp.float32)
        return lax.dot_general(
            oh, w_ref[...], (((0,), (0,)), ((), ())),
            preferred_element_type=jnp.float32,
        )

    ex = part(ix_ref, wx_ref)
    ey = part(iy_ref, wy_ref)
    ez = part(iz_ref, wz_ref)
    xb = x_ref[0]
    o_ref[0, :, 0:DPART] = xb[:, 0:DPART] + ex
    o_ref[0, :, DPART:2 * DPART] = xb[:, DPART:2 * DPART] + ey
    o_ref[0, :, 2 * DPART:D_MODEL] = xb[:, 2 * DPART:D_MODEL] + ez


def kernel(x, src_tgt, src_pos_x, src_pos_y, src_pos_z, Wx, Wy, Wz):
    del src_tgt
    B = x.shape[0]
    ix = src_pos_x.reshape(N_SBLK, 1, S_BLK)
    iy = src_pos_y.reshape(N_SBLK, 1, S_BLK)
    iz = src_pos_z.reshape(N_SBLK, 1, S_BLK)

    idx_spec = pl.BlockSpec((1, 1, S_BLK), lambda i, j: (i, 0, 0))
    tab_spec = pl.BlockSpec((32, DPART), lambda i, j: (0, 0))
    x_spec = pl.BlockSpec((1, S_BLK, D_MODEL), lambda i, j: (j, i, 0))

    return pl.pallas_call(
        _body,
        grid=(N_SBLK, B),
        in_specs=[idx_spec, idx_spec, idx_spec, x_spec, tab_spec, tab_spec,
                  tab_spec],
        out_specs=x_spec,
        out_shape=jax.ShapeDtypeStruct(x.shape, x.dtype),
    )(ix, iy, iz, x, Wx, Wy, Wz)
